# baseline (speedup 1.0000x reference)
import functools

import jax
import jax.numpy as jnp
from jax import lax
from jax.experimental import pallas as pl
from jax.experimental.pallas import tpu as pltpu
from jax.experimental.pallas import tpu_sc as plsc

NU = 10000
NI = 10000
E = 320000
EQ = 100000
D = 128
H = 64

NC = 2
NS = 16
NW = NC * NS
L = 16
CHUNK = E // NW

EQ_ROWS = 782
EQ_PAD = EQ_ROWS * 128


def _hist_body(eu_hbm, ei_hbm, out_u, out_i, idx_v, cu_v, ci_v):
    wid = lax.axis_index("c") * NS + lax.axis_index("s")
    zeros16 = jnp.zeros((L,), jnp.float32)
    ones16 = jnp.ones((L,), jnp.float32)

    def zero_body(i, carry):
        cu_v[pl.ds(i * L, L)] = zeros16
        ci_v[pl.ds(i * L, L)] = zeros16
        return carry

    lax.fori_loop(0, NU // L, zero_body, 0, unroll=8)

    base = wid * CHUNK
    pltpu.sync_copy(eu_hbm.at[0, pl.ds(base, CHUNK)], idx_v)

    def add_u(i, carry):
        iv = idx_v[pl.ds(i * L, L)]
        plsc.addupdate_scatter(cu_v, [iv], ones16)
        return carry

    lax.fori_loop(0, CHUNK // L, add_u, 0, unroll=4)

    pltpu.sync_copy(ei_hbm.at[0, pl.ds(base, CHUNK)], idx_v)

    def add_i(i, carry):
        iv = idx_v[pl.ds(i * L, L)]
        plsc.addupdate_scatter(ci_v, [iv], ones16)
        return carry

    lax.fori_loop(0, CHUNK // L, add_i, 0, unroll=4)

    pltpu.sync_copy(cu_v, out_u.at[wid])
    pltpu.sync_copy(ci_v, out_i.at[wid])


@functools.cache
def _hist():
    return pl.kernel(
        _hist_body,
        mesh=plsc.VectorSubcoreMesh(core_axis_name="c", subcore_axis_name="s"),
        out_type=[
            jax.ShapeDtypeStruct((NW, NU), jnp.float32),
            jax.ShapeDtypeStruct((NW, NI), jnp.float32),
        ],
        scratch_types=[
            pltpu.VMEM((CHUNK,), jnp.int32),
            pltpu.VMEM((NU,), jnp.float32),
            pltpu.VMEM((NI,), jnp.float32),
        ],
        compiler_params=pltpu.CompilerParams(
            use_tc_tiling_on_sc=False,
            needs_layout_passes=False,
        ),
    )


def _dense_body(pu_ref, pi_ref, xu_ref, xi_ref, weu_ref, beu_ref, wei_ref,
                bei_ref, w1_ref, b1_ref, w2_ref, b2_ref, src_ref, dst_ref,
                out_ref):
    inv_e = jnp.float32(1.0 / E)
    cu = jnp.sum(pu_ref[...], axis=0)
    ci = jnp.sum(pi_ref[...], axis=0)

    mean_user = jnp.sum(xu_ref[...] * cu[:, None], axis=0, keepdims=True) * inv_e
    mean_item = jnp.sum(xi_ref[...] * ci[:, None], axis=0, keepdims=True) * inv_e

    dot = functools.partial(
        lax.dot_general,
        dimension_numbers=(((1,), (0,)), ((), ())),
        preferred_element_type=jnp.float32,
        precision=lax.Precision.HIGHEST,
    )
    msg_i1 = dot(mean_user, weu_ref[...]) + beu_ref[...]
    msg_u1 = dot(mean_item, wei_ref[...]) + bei_ref[...]

    p_u = lax.slice(ci, (0,), (1,)).reshape(1, 1) * inv_e
    p_i = lax.slice(cu, (0,), (1,)).reshape(1, 1) * inv_e
    scale = p_u * p_i
    u_vec = scale * jnp.maximum(msg_u1, 0.0)
    i_vec = scale * jnp.maximum(msg_i1, 0.0)

    z = jnp.zeros((1, H), jnp.float32)
    combos = jnp.concatenate(
        [
            jnp.concatenate([z, z], axis=1),
            jnp.concatenate([z, i_vec], axis=1),
            jnp.concatenate([u_vec, z], axis=1),
            jnp.concatenate([u_vec, i_vec], axis=1),
        ],
        axis=0,
    )
    hid = jnp.maximum(dot(combos, w1_ref[...]) + b1_ref[...], 0.0)
    vals = jax.nn.sigmoid(dot(hid, w2_ref[...]) + b2_ref[...])

    v00 = lax.slice(vals, (0, 0), (1, 1))
    v01 = lax.slice(vals, (1, 0), (2, 1))
    v10 = lax.slice(vals, (2, 0), (3, 1))
    v11 = lax.slice(vals, (3, 0), (4, 1))

    s_mask = src_ref[...] == 0
    d_mask = dst_ref[...] == 0
    out_ref[...] = jnp.where(
        s_mask,
        jnp.where(d_mask, v11, v10),
        jnp.where(d_mask, v01, v00),
    )


_dense = pl.pallas_call(
    _dense_body,
    out_shape=jax.ShapeDtypeStruct((EQ_ROWS, 128), jnp.float32),
)


def kernel(x_user, x_item, edge_index_u2i, edge_index_i2u, edge_label_index,
           W_emb_user, b_emb_user, W_emb_item, b_emb_item, W1, b1, W2, b2):
    part_u, part_i = _hist()(edge_index_u2i[0:1].astype(jnp.int32),
                             edge_index_i2u[0:1].astype(jnp.int32))

    eli = edge_label_index.astype(jnp.int32)
    pad = jnp.ones((2, EQ_PAD - EQ), jnp.int32)
    eli_p = jnp.concatenate([eli, pad], axis=1).reshape(2, EQ_ROWS, 128)

    out2d = _dense(
        part_u, part_i,
        x_user, x_item,
        W_emb_user, b_emb_user.reshape(1, H),
        W_emb_item, b_emb_item.reshape(1, H),
        W1, b1.reshape(1, H),
        W2, b2.reshape(1, 1),
        eli_p[0], eli_p[1],
    )
    return out2d.reshape(EQ_PAD)[:EQ]


# speedup vs baseline: 1.1568x; 1.1568x over previous
import functools

import jax
import jax.numpy as jnp
from jax import lax
from jax.experimental import pallas as pl
from jax.experimental.pallas import tpu as pltpu
from jax.experimental.pallas import tpu_sc as plsc

NU = 10000
NI = 10000
E = 320000
EQ = 100000
D = 128
H = 64

NC = 2
NS = 16
NW = NC * NS
L = 16
CHUNK = E // NW

EQ_ROWS = 782
EQ_PAD = EQ_ROWS * 128


def _hist_body(eu_hbm, ei_hbm, out_u, out_i, idx_v, cu_v, ci_v):
    wid = lax.axis_index("c") * NS + lax.axis_index("s")
    zeros16 = jnp.zeros((L,), jnp.float32)
    ones16 = jnp.ones((L,), jnp.float32)

    def zero_body(i, carry):
        cu_v[pl.ds(i * L, L)] = zeros16
        ci_v[pl.ds(i * L, L)] = zeros16
        return carry

    lax.fori_loop(0, NU // L, zero_body, 0, unroll=8)

    base = wid * CHUNK
    pltpu.sync_copy(eu_hbm.at[0, pl.ds(base, CHUNK)], idx_v)

    def add_u(i, carry):
        iv = idx_v[pl.ds(i * L, L)]
        plsc.addupdate_scatter(cu_v, [iv], ones16)
        return carry

    lax.fori_loop(0, CHUNK // L, add_u, 0, unroll=4)

    pltpu.sync_copy(ei_hbm.at[0, pl.ds(base, CHUNK)], idx_v)

    def add_i(i, carry):
        iv = idx_v[pl.ds(i * L, L)]
        plsc.addupdate_scatter(ci_v, [iv], ones16)
        return carry

    lax.fori_loop(0, CHUNK // L, add_i, 0, unroll=4)

    pltpu.sync_copy(cu_v, out_u.at[wid])
    pltpu.sync_copy(ci_v, out_i.at[wid])


@functools.cache
def _hist():
    return pl.kernel(
        _hist_body,
        mesh=plsc.VectorSubcoreMesh(core_axis_name="c", subcore_axis_name="s"),
        out_type=[
            jax.ShapeDtypeStruct((NW, NU), jnp.float32),
            jax.ShapeDtypeStruct((NW, NI), jnp.float32),
        ],
        scratch_types=[
            pltpu.VMEM((CHUNK,), jnp.int32),
            pltpu.VMEM((NU,), jnp.float32),
            pltpu.VMEM((NI,), jnp.float32),
        ],
        compiler_params=pltpu.CompilerParams(
            use_tc_tiling_on_sc=False,
            needs_layout_passes=False,
        ),
    )


_dot = functools.partial(
    lax.dot_general,
    dimension_numbers=(((1,), (0,)), ((), ())),
    preferred_element_type=jnp.float32,
    precision=lax.Precision.HIGHEST,
)


def _proj_body(xu_ref, weu_ref, xi_ref, wei_ref, yu_ref, yi_ref):
    yu_ref[...] = _dot(xu_ref[...], weu_ref[...])
    yi_ref[...] = _dot(xi_ref[...], wei_ref[...])


_proj = pl.pallas_call(
    _proj_body,
    out_shape=[
        jax.ShapeDtypeStruct((NU, H), jnp.float32),
        jax.ShapeDtypeStruct((NI, H), jnp.float32),
    ],
)


def _dense_body(pu_ref, pi_ref, yu_ref, yi_ref, beu_ref,
                bei_ref, w1_ref, b1_ref, w2_ref, b2_ref, src_ref, dst_ref,
                out_ref):
    inv_e = jnp.float32(1.0 / E)
    cu = jnp.sum(pu_ref[...], axis=0)
    ci = jnp.sum(pi_ref[...], axis=0)

    msg_i1 = (jnp.sum(yu_ref[...] * cu[:, None], axis=0, keepdims=True)
              * inv_e + beu_ref[...])
    msg_u1 = (jnp.sum(yi_ref[...] * ci[:, None], axis=0, keepdims=True)
              * inv_e + bei_ref[...])

    p_u = lax.slice(ci, (0,), (1,)).reshape(1, 1) * inv_e
    p_i = lax.slice(cu, (0,), (1,)).reshape(1, 1) * inv_e
    scale = p_u * p_i
    u_vec = scale * jnp.maximum(msg_u1, 0.0)
    i_vec = scale * jnp.maximum(msg_i1, 0.0)

    z = jnp.zeros((1, H), jnp.float32)
    combos = jnp.concatenate(
        [
            jnp.concatenate([z, z], axis=1),
            jnp.concatenate([z, i_vec], axis=1),
            jnp.concatenate([u_vec, z], axis=1),
            jnp.concatenate([u_vec, i_vec], axis=1),
        ],
        axis=0,
    )
    hid = jnp.maximum(_dot(combos, w1_ref[...]) + b1_ref[...], 0.0)
    vals = jax.nn.sigmoid(_dot(hid, w2_ref[...]) + b2_ref[...])

    v00 = lax.slice(vals, (0, 0), (1, 1))
    v01 = lax.slice(vals, (1, 0), (2, 1))
    v10 = lax.slice(vals, (2, 0), (3, 1))
    v11 = lax.slice(vals, (3, 0), (4, 1))

    s_mask = src_ref[...] == 0
    d_mask = dst_ref[...] == 0
    out_ref[...] = jnp.where(
        s_mask,
        jnp.where(d_mask, v11, v10),
        jnp.where(d_mask, v01, v00),
    )


_dense = pl.pallas_call(
    _dense_body,
    out_shape=jax.ShapeDtypeStruct((EQ_ROWS, 128), jnp.float32),
)


def kernel(x_user, x_item, edge_index_u2i, edge_index_i2u, edge_label_index,
           W_emb_user, b_emb_user, W_emb_item, b_emb_item, W1, b1, W2, b2):
    part_u, part_i = _hist()(edge_index_u2i.astype(jnp.int32),
                             edge_index_i2u.astype(jnp.int32))

    eli = edge_label_index.astype(jnp.int32)
    pad = jnp.ones((2, EQ_PAD - EQ), jnp.int32)
    eli_p = jnp.concatenate([eli, pad], axis=1).reshape(2, EQ_ROWS, 128)

    y_u, y_i = _proj(x_user, W_emb_user, x_item, W_emb_item)
    out2d = _dense(
        part_u, part_i,
        y_u, y_i,
        b_emb_user.reshape(1, H),
        b_emb_item.reshape(1, H),
        W1, b1.reshape(1, H),
        W2, b2.reshape(1, 1),
        eli_p[0], eli_p[1],
    )
    return out2d.reshape(EQ_PAD)[:EQ]


# speedup vs baseline: 1.3082x; 1.1309x over previous
import functools

import jax
import jax.numpy as jnp
from jax import lax
from jax.experimental import pallas as pl
from jax.experimental.pallas import tpu as pltpu
from jax.experimental.pallas import tpu_sc as plsc

NU = 10000
NI = 10000
E = 320000
EQ = 100000
D = 128
H = 64

NC = 2
NS = 16
NW = NC * NS
L = 16
CHUNK = E // NW
NUP = 10240
COLS = NUP // NS

EQ_ROWS = 782
EQ_PAD = EQ_ROWS * 128


def _hist_body(eu_hbm, ei_hbm, out_u, out_i, idx_v, cu_v, ci_v, red_v,
               sh_u, sh_i):
    cid = lax.axis_index("c")
    sid = lax.axis_index("s")
    wid = cid * NS + sid
    zeros16 = jnp.zeros((L,), jnp.float32)
    ones16 = jnp.ones((L,), jnp.float32)

    def zero_body(i, carry):
        cu_v[pl.ds(i * L, L)] = zeros16
        ci_v[pl.ds(i * L, L)] = zeros16
        return carry

    lax.fori_loop(0, NUP // L, zero_body, 0, unroll=8)

    base = wid * CHUNK
    pltpu.sync_copy(eu_hbm.at[0, pl.ds(base, CHUNK)], idx_v)

    def add_u(i, carry):
        iv = idx_v[pl.ds(i * L, L)]
        plsc.addupdate_scatter(cu_v, [iv], ones16)
        return carry

    lax.fori_loop(0, CHUNK // L, add_u, 0, unroll=4)

    pltpu.sync_copy(ei_hbm.at[0, pl.ds(base, CHUNK)], idx_v)

    def add_i(i, carry):
        iv = idx_v[pl.ds(i * L, L)]
        plsc.addupdate_scatter(ci_v, [iv], ones16)
        return carry

    lax.fori_loop(0, CHUNK // L, add_i, 0, unroll=4)

    pltpu.sync_copy(cu_v, sh_u.at[sid])
    pltpu.sync_copy(ci_v, sh_i.at[sid])
    plsc.subcore_barrier()

    col = sid * COLS
    for r in range(NS):
        pltpu.sync_copy(sh_u.at[r, pl.ds(col, COLS)],
                        red_v.at[pl.ds(r * COLS, COLS)])

    def red_u(j, carry):
        acc = red_v[pl.ds(j * L, L)]
        for r in range(1, NS):
            acc = acc + red_v[pl.ds(r * COLS + j * L, L)]
        cu_v[pl.ds(j * L, L)] = acc
        return carry

    lax.fori_loop(0, COLS // L, red_u, 0, unroll=4)
    pltpu.sync_copy(cu_v.at[pl.ds(0, COLS)], out_u.at[cid, pl.ds(col, COLS)])

    for r in range(NS):
        pltpu.sync_copy(sh_i.at[r, pl.ds(col, COLS)],
                        red_v.at[pl.ds(r * COLS, COLS)])

    def red_i(j, carry):
        acc = red_v[pl.ds(j * L, L)]
        for r in range(1, NS):
            acc = acc + red_v[pl.ds(r * COLS + j * L, L)]
        ci_v[pl.ds(j * L, L)] = acc
        return carry

    lax.fori_loop(0, COLS // L, red_i, 0, unroll=4)
    pltpu.sync_copy(ci_v.at[pl.ds(0, COLS)], out_i.at[cid, pl.ds(col, COLS)])


@functools.cache
def _hist():
    return pl.kernel(
        _hist_body,
        mesh=plsc.VectorSubcoreMesh(core_axis_name="c", subcore_axis_name="s"),
        out_type=[
            jax.ShapeDtypeStruct((NC, NUP), jnp.float32),
            jax.ShapeDtypeStruct((NC, NUP), jnp.float32),
        ],
        scratch_types=[
            pltpu.VMEM((CHUNK,), jnp.int32),
            pltpu.VMEM((NUP,), jnp.float32),
            pltpu.VMEM((NUP,), jnp.float32),
            pltpu.VMEM((NS * COLS,), jnp.float32),
            pltpu.VMEM_SHARED((NS, NUP), jnp.float32),
            pltpu.VMEM_SHARED((NS, NUP), jnp.float32),
        ],
        compiler_params=pltpu.CompilerParams(
            use_tc_tiling_on_sc=False,
            needs_layout_passes=False,
        ),
    )


_dot = functools.partial(
    lax.dot_general,
    dimension_numbers=(((1,), (0,)), ((), ())),
    preferred_element_type=jnp.float32,
    precision=lax.Precision.HIGHEST,
)


def _dense_body(pu_ref, pi_ref, xu_ref, xi_ref, weu_ref, beu_ref, wei_ref,
                bei_ref, w1_ref, b1_ref, w2_ref, b2_ref, src_ref, dst_ref,
                out_ref):
    inv_e = jnp.float32(1.0 / E)
    cu = jnp.sum(pu_ref[...], axis=0)
    ci = jnp.sum(pi_ref[...], axis=0)
    cu_t = lax.slice(cu, (0,), (NU,))
    ci_t = lax.slice(ci, (0,), (NI,))

    mean_user = jnp.sum(xu_ref[...] * cu_t[:, None], axis=0,
                        keepdims=True) * inv_e
    mean_item = jnp.sum(xi_ref[...] * ci_t[:, None], axis=0,
                        keepdims=True) * inv_e

    msg_i1 = _dot(mean_user, weu_ref[...]) + beu_ref[...]
    msg_u1 = _dot(mean_item, wei_ref[...]) + bei_ref[...]

    p_u = lax.slice(ci, (0,), (1,)).reshape(1, 1) * inv_e
    p_i = lax.slice(cu, (0,), (1,)).reshape(1, 1) * inv_e
    scale = p_u * p_i
    u_vec = scale * jnp.maximum(msg_u1, 0.0)
    i_vec = scale * jnp.maximum(msg_i1, 0.0)

    z = jnp.zeros((1, H), jnp.float32)
    combos = jnp.concatenate(
        [
            jnp.concatenate([z, z], axis=1),
            jnp.concatenate([z, i_vec], axis=1),
            jnp.concatenate([u_vec, z], axis=1),
            jnp.concatenate([u_vec, i_vec], axis=1),
        ],
        axis=0,
    )
    hid = jnp.maximum(_dot(combos, w1_ref[...]) + b1_ref[...], 0.0)
    vals = jax.nn.sigmoid(_dot(hid, w2_ref[...]) + b2_ref[...])

    v00 = lax.slice(vals, (0, 0), (1, 1))
    v01 = lax.slice(vals, (1, 0), (2, 1))
    v10 = lax.slice(vals, (2, 0), (3, 1))
    v11 = lax.slice(vals, (3, 0), (4, 1))

    s_mask = src_ref[...] == 0
    d_mask = dst_ref[...] == 0
    out_ref[...] = jnp.where(
        s_mask,
        jnp.where(d_mask, v11, v10),
        jnp.where(d_mask, v01, v00),
    )


_dense = pl.pallas_call(
    _dense_body,
    out_shape=jax.ShapeDtypeStruct((EQ_ROWS, 128), jnp.float32),
)


def kernel(x_user, x_item, edge_index_u2i, edge_index_i2u, edge_label_index,
           W_emb_user, b_emb_user, W_emb_item, b_emb_item, W1, b1, W2, b2):
    part_u, part_i = _hist()(edge_index_u2i.astype(jnp.int32),
                             edge_index_i2u.astype(jnp.int32))

    eli = edge_label_index.astype(jnp.int32)
    pad = jnp.ones((2, EQ_PAD - EQ), jnp.int32)
    eli_p = jnp.concatenate([eli, pad], axis=1).reshape(2, EQ_ROWS, 128)

    out2d = _dense(
        part_u, part_i,
        x_user, x_item,
        W_emb_user, b_emb_user.reshape(1, H),
        W_emb_item, b_emb_item.reshape(1, H),
        W1, b1.reshape(1, H),
        W2, b2.reshape(1, 1),
        eli_p[0], eli_p[1],
    )
    return out2d.reshape(EQ_PAD)[:EQ]


# speedup vs baseline: 1.5646x; 1.1960x over previous
import functools

import jax
import jax.numpy as jnp
from jax import lax
from jax.experimental import pallas as pl
from jax.experimental.pallas import tpu as pltpu
from jax.experimental.pallas import tpu_sc as plsc

NU = 10000
NI = 10000
E = 320000
EQ = 100000
D = 128
H = 64

NC = 2
NS = 16
NW = NC * NS
L = 16
CHUNK = E // NW
NUP = 10240

EQ_ROWS = 782
EQ_PAD = EQ_ROWS * 128


def _hist_body(eu_hbm, ei_hbm, out_u, out_i, idx_v, cu_v, ci_v):
    wid = lax.axis_index("c") * NS + lax.axis_index("s")
    zeros16 = jnp.zeros((L,), jnp.float32)
    ones16 = jnp.ones((L,), jnp.float32)

    def zero_body(i, carry):
        cu_v[pl.ds(i * L, L)] = zeros16
        ci_v[pl.ds(i * L, L)] = zeros16
        return carry

    lax.fori_loop(0, NUP // L, zero_body, 0, unroll=8)

    base = wid * CHUNK
    pltpu.sync_copy(eu_hbm.at[0, pl.ds(base, CHUNK)], idx_v)

    def add_u(i, carry):
        iv = idx_v[pl.ds(i * L, L)]
        plsc.addupdate_scatter(cu_v, [iv], ones16)
        return carry

    lax.fori_loop(0, CHUNK // L, add_u, 0, unroll=4)

    pltpu.sync_copy(ei_hbm.at[0, pl.ds(base, CHUNK)], idx_v)

    def add_i(i, carry):
        iv = idx_v[pl.ds(i * L, L)]
        plsc.addupdate_scatter(ci_v, [iv], ones16)
        return carry

    lax.fori_loop(0, CHUNK // L, add_i, 0, unroll=4)

    pltpu.sync_copy(cu_v, out_u.at[pl.ds(wid * NUP, NUP)])
    pltpu.sync_copy(ci_v, out_i.at[pl.ds(wid * NUP, NUP)])


@functools.cache
def _hist():
    return pl.kernel(
        _hist_body,
        mesh=plsc.VectorSubcoreMesh(core_axis_name="c", subcore_axis_name="s"),
        out_type=[
            jax.ShapeDtypeStruct((NW * NUP,), jnp.float32),
            jax.ShapeDtypeStruct((NW * NUP,), jnp.float32),
        ],
        scratch_types=[
            pltpu.VMEM((CHUNK,), jnp.int32),
            pltpu.VMEM((NUP,), jnp.float32),
            pltpu.VMEM((NUP,), jnp.float32),
        ],
        compiler_params=pltpu.CompilerParams(
            use_tc_tiling_on_sc=False,
            needs_layout_passes=False,
        ),
    )


_dot = functools.partial(
    lax.dot_general,
    dimension_numbers=(((1,), (0,)), ((), ())),
    preferred_element_type=jnp.float32,
    precision=lax.Precision.HIGHEST,
)


def _dense_body(pu_ref, pi_ref, xu_ref, xi_ref, weu_ref, beu_ref, wei_ref,
                bei_ref, w1_ref, b1_ref, w2_ref, b2_ref, src_ref, dst_ref,
                out_ref):
    inv_e = jnp.float32(1.0 / E)
    cu = pu_ref[pl.ds(0, NU)]
    ci = pi_ref[pl.ds(0, NI)]
    for w in range(1, NW):
        cu = cu + pu_ref[pl.ds(w * NUP, NU)]
        ci = ci + pi_ref[pl.ds(w * NUP, NI)]

    mean_user = jnp.sum(xu_ref[...] * cu[:, None], axis=0,
                        keepdims=True) * inv_e
    mean_item = jnp.sum(xi_ref[...] * ci[:, None], axis=0,
                        keepdims=True) * inv_e

    msg_i1 = _dot(mean_user, weu_ref[...]) + beu_ref[...]
    msg_u1 = _dot(mean_item, wei_ref[...]) + bei_ref[...]

    p_u = lax.slice(ci, (0,), (1,)).reshape(1, 1) * inv_e
    p_i = lax.slice(cu, (0,), (1,)).reshape(1, 1) * inv_e
    scale = p_u * p_i
    u_vec = scale * jnp.maximum(msg_u1, 0.0)
    i_vec = scale * jnp.maximum(msg_i1, 0.0)

    z = jnp.zeros((1, H), jnp.float32)
    combos = jnp.concatenate(
        [
            jnp.concatenate([z, z], axis=1),
            jnp.concatenate([z, i_vec], axis=1),
            jnp.concatenate([u_vec, z], axis=1),
            jnp.concatenate([u_vec, i_vec], axis=1),
        ],
        axis=0,
    )
    hid = jnp.maximum(_dot(combos, w1_ref[...]) + b1_ref[...], 0.0)
    vals = jax.nn.sigmoid(_dot(hid, w2_ref[...]) + b2_ref[...])

    v00 = lax.slice(vals, (0, 0), (1, 1))
    v01 = lax.slice(vals, (1, 0), (2, 1))
    v10 = lax.slice(vals, (2, 0), (3, 1))
    v11 = lax.slice(vals, (3, 0), (4, 1))

    s_mask = src_ref[...] == 0
    d_mask = dst_ref[...] == 0
    out_ref[...] = jnp.where(
        s_mask,
        jnp.where(d_mask, v11, v10),
        jnp.where(d_mask, v01, v00),
    )


_dense = pl.pallas_call(
    _dense_body,
    out_shape=jax.ShapeDtypeStruct((EQ_ROWS, 128), jnp.float32),
)


def kernel(x_user, x_item, edge_index_u2i, edge_index_i2u, edge_label_index,
           W_emb_user, b_emb_user, W_emb_item, b_emb_item, W1, b1, W2, b2):
    part_u, part_i = _hist()(edge_index_u2i.astype(jnp.int32),
                             edge_index_i2u.astype(jnp.int32))

    eli = edge_label_index.astype(jnp.int32)
    pad = jnp.ones((2, EQ_PAD - EQ), jnp.int32)
    eli_p = jnp.concatenate([eli, pad], axis=1).reshape(2, EQ_ROWS, 128)

    out2d = _dense(
        part_u, part_i,
        x_user, x_item,
        W_emb_user, b_emb_user.reshape(1, H),
        W_emb_item, b_emb_item.reshape(1, H),
        W1, b1.reshape(1, H),
        W2, b2.reshape(1, 1),
        eli_p[0], eli_p[1],
    )
    return out2d.reshape(EQ_PAD)[:EQ]


# speedup vs baseline: 1.6710x; 1.0680x over previous
import functools

import jax
import jax.numpy as jnp
from jax import lax
from jax.experimental import pallas as pl
from jax.experimental.pallas import tpu as pltpu
from jax.experimental.pallas import tpu_sc as plsc

NU = 10000
NI = 10000
E = 320000
EQ = 100000
D = 128
H = 64

NC = 2
NS = 16
NW = NC * NS
L = 16
CHUNK = E // NW
NUP = 10240

EQ_ROWS = 782
EQ_PAD = EQ_ROWS * 128


WIN = 9984
REM = E - NW * WIN


def _hist_body(eu_hbm, ei_hbm, out_u, out_i, idx_v, rem_v, cu_v, ci_v):
    wid = lax.axis_index("c") * NS + lax.axis_index("s")
    zeros16 = jnp.zeros((L,), jnp.float32)
    ones16 = jnp.ones((L,), jnp.float32)

    def zero_body(i, carry):
        cu_v[pl.ds(i * L, L)] = zeros16
        ci_v[pl.ds(i * L, L)] = zeros16
        return carry

    lax.fori_loop(0, NUP // L, zero_body, 0, unroll=8)

    base = wid * WIN
    pltpu.sync_copy(eu_hbm.at[:, pl.ds(base, WIN)], idx_v)

    def add_u(i, carry):
        iv = idx_v[0, pl.ds(i * L, L)]
        plsc.addupdate_scatter(cu_v, [iv], ones16)
        return carry

    lax.fori_loop(0, WIN // L, add_u, 0, unroll=4)

    pltpu.sync_copy(ei_hbm.at[:, pl.ds(base, WIN)], idx_v)

    def add_i(i, carry):
        iv = idx_v[0, pl.ds(i * L, L)]
        plsc.addupdate_scatter(ci_v, [iv], ones16)
        return carry

    lax.fori_loop(0, WIN // L, add_i, 0, unroll=4)

    @pl.when(wid == 0)
    def _():
        pltpu.sync_copy(eu_hbm.at[:, pl.ds(NW * WIN, REM)], rem_v)

        def add_ru(i, carry):
            iv = rem_v[0, pl.ds(i * L, L)]
            plsc.addupdate_scatter(cu_v, [iv], ones16)
            return carry

        lax.fori_loop(0, REM // L, add_ru, 0, unroll=4)

        pltpu.sync_copy(ei_hbm.at[:, pl.ds(NW * WIN, REM)], rem_v)

        def add_ri(i, carry):
            iv = rem_v[0, pl.ds(i * L, L)]
            plsc.addupdate_scatter(ci_v, [iv], ones16)
            return carry

        lax.fori_loop(0, REM // L, add_ri, 0, unroll=4)

    pltpu.sync_copy(cu_v, out_u.at[pl.ds(wid * NUP, NUP)])
    pltpu.sync_copy(ci_v, out_i.at[pl.ds(wid * NUP, NUP)])


@functools.cache
def _hist():
    return pl.kernel(
        _hist_body,
        mesh=plsc.VectorSubcoreMesh(core_axis_name="c", subcore_axis_name="s"),
        out_type=[
            jax.ShapeDtypeStruct((NW * NUP,), jnp.float32),
            jax.ShapeDtypeStruct((NW * NUP,), jnp.float32),
        ],
        scratch_types=[
            pltpu.VMEM((2, WIN), jnp.int32),
            pltpu.VMEM((2, REM), jnp.int32),
            pltpu.VMEM((NUP,), jnp.float32),
            pltpu.VMEM((NUP,), jnp.float32),
        ],
        compiler_params=pltpu.CompilerParams(
            use_tc_tiling_on_sc=True,
            needs_layout_passes=False,
        ),
    )


_dot = functools.partial(
    lax.dot_general,
    dimension_numbers=(((1,), (0,)), ((), ())),
    preferred_element_type=jnp.float32,
    precision=lax.Precision.HIGHEST,
)


def _dense_body(pu_ref, pi_ref, xu_ref, xi_ref, weu_ref, beu_ref, wei_ref,
                bei_ref, w1_ref, b1_ref, w2_ref, b2_ref, src_ref, dst_ref,
                out_ref):
    inv_e = jnp.float32(1.0 / E)
    cu = pu_ref[pl.ds(0, NU)]
    ci = pi_ref[pl.ds(0, NI)]
    for w in range(1, NW):
        cu = cu + pu_ref[pl.ds(w * NUP, NU)]
        ci = ci + pi_ref[pl.ds(w * NUP, NI)]

    mean_user = jnp.sum(xu_ref[...] * cu[:, None], axis=0,
                        keepdims=True) * inv_e
    mean_item = jnp.sum(xi_ref[...] * ci[:, None], axis=0,
                        keepdims=True) * inv_e

    msg_i1 = _dot(mean_user, weu_ref[...]) + beu_ref[...]
    msg_u1 = _dot(mean_item, wei_ref[...]) + bei_ref[...]

    p_u = lax.slice(ci, (0,), (1,)).reshape(1, 1) * inv_e
    p_i = lax.slice(cu, (0,), (1,)).reshape(1, 1) * inv_e
    scale = p_u * p_i
    u_vec = scale * jnp.maximum(msg_u1, 0.0)
    i_vec = scale * jnp.maximum(msg_i1, 0.0)

    z = jnp.zeros((1, H), jnp.float32)
    combos = jnp.concatenate(
        [
            jnp.concatenate([z, z], axis=1),
            jnp.concatenate([z, i_vec], axis=1),
            jnp.concatenate([u_vec, z], axis=1),
            jnp.concatenate([u_vec, i_vec], axis=1),
        ],
        axis=0,
    )
    hid = jnp.maximum(_dot(combos, w1_ref[...]) + b1_ref[...], 0.0)
    vals = jax.nn.sigmoid(_dot(hid, w2_ref[...]) + b2_ref[...])

    v00 = lax.slice(vals, (0, 0), (1, 1))
    v01 = lax.slice(vals, (1, 0), (2, 1))
    v10 = lax.slice(vals, (2, 0), (3, 1))
    v11 = lax.slice(vals, (3, 0), (4, 1))

    s_mask = src_ref[...] == 0
    d_mask = dst_ref[...] == 0
    out_ref[...] = jnp.where(
        s_mask,
        jnp.where(d_mask, v11, v10),
        jnp.where(d_mask, v01, v00),
    )


_dense = pl.pallas_call(
    _dense_body,
    out_shape=jax.ShapeDtypeStruct((EQ_ROWS, 128), jnp.float32),
)


def kernel(x_user, x_item, edge_index_u2i, edge_index_i2u, edge_label_index,
           W_emb_user, b_emb_user, W_emb_item, b_emb_item, W1, b1, W2, b2):
    part_u, part_i = _hist()(edge_index_u2i.astype(jnp.int32),
                             edge_index_i2u.astype(jnp.int32))

    eli = edge_label_index.astype(jnp.int32)
    pad = jnp.ones((2, EQ_PAD - EQ), jnp.int32)
    eli_p = jnp.concatenate([eli, pad], axis=1).reshape(2, EQ_ROWS, 128)

    out2d = _dense(
        part_u, part_i,
        x_user, x_item,
        W_emb_user, b_emb_user.reshape(1, H),
        W_emb_item, b_emb_item.reshape(1, H),
        W1, b1.reshape(1, H),
        W2, b2.reshape(1, 1),
        eli_p[0], eli_p[1],
    )
    return out2d.reshape(EQ_PAD)[:EQ]


# speedup vs baseline: 1.9764x; 1.1828x over previous
import functools

import jax
import jax.numpy as jnp
from jax import lax
from jax.experimental import pallas as pl
from jax.experimental.pallas import tpu as pltpu
from jax.experimental.pallas import tpu_sc as plsc

NU = 10000
NI = 10000
E = 320000
EQ = 100000
D = 128
H = 64

NC = 2
NS = 16
NW = NC * NS
L = 16
CHUNK = E // NW
NUP = 10240

EQ_ROWS = 782
EQ_PAD = EQ_ROWS * 128


WIN = 9984
REM = E - NW * WIN


def _hist_body(eu_hbm, ei_hbm, out_u, out_i, idx_u, idx_i, rem_v, cu_v, ci_v,
               sem_u, sem_i):
    wid = lax.axis_index("c") * NS + lax.axis_index("s")
    zeros16 = jnp.zeros((L,), jnp.float32)
    ones16 = jnp.ones((L,), jnp.float32)

    base = wid * WIN
    cp_u = pltpu.async_copy(eu_hbm.at[:, pl.ds(base, WIN)], idx_u, sem_u)
    cp_i = pltpu.async_copy(ei_hbm.at[:, pl.ds(base, WIN)], idx_i, sem_i)

    def zero_body(i, carry):
        cu_v[pl.ds(i * L, L)] = zeros16
        ci_v[pl.ds(i * L, L)] = zeros16
        return carry

    lax.fori_loop(0, NUP // L, zero_body, 0, unroll=8)

    cp_u.wait()

    @plsc.parallel_loop(0, WIN // L, unroll=4)
    def _(i):
        iv = idx_u[0, pl.ds(i * L, L)]
        plsc.addupdate_scatter(cu_v, [iv], ones16)

    cp_i.wait()

    @plsc.parallel_loop(0, WIN // L, unroll=4)
    def _(i):
        iv = idx_i[0, pl.ds(i * L, L)]
        plsc.addupdate_scatter(ci_v, [iv], ones16)

    @pl.when(wid == 0)
    def _():
        pltpu.sync_copy(eu_hbm.at[:, pl.ds(NW * WIN, REM)], rem_v)

        def add_ru(i, carry):
            iv = rem_v[0, pl.ds(i * L, L)]
            plsc.addupdate_scatter(cu_v, [iv], ones16)
            return carry

        lax.fori_loop(0, REM // L, add_ru, 0, unroll=4)

        pltpu.sync_copy(ei_hbm.at[:, pl.ds(NW * WIN, REM)], rem_v)

        def add_ri(i, carry):
            iv = rem_v[0, pl.ds(i * L, L)]
            plsc.addupdate_scatter(ci_v, [iv], ones16)
            return carry

        lax.fori_loop(0, REM // L, add_ri, 0, unroll=4)

    pltpu.sync_copy(cu_v, out_u.at[pl.ds(wid * NUP, NUP)])
    pltpu.sync_copy(ci_v, out_i.at[pl.ds(wid * NUP, NUP)])


@functools.cache
def _hist():
    return pl.kernel(
        _hist_body,
        mesh=plsc.VectorSubcoreMesh(core_axis_name="c", subcore_axis_name="s"),
        out_type=[
            jax.ShapeDtypeStruct((NW * NUP,), jnp.float32),
            jax.ShapeDtypeStruct((NW * NUP,), jnp.float32),
        ],
        scratch_types=[
            pltpu.VMEM((2, WIN), jnp.int32),
            pltpu.VMEM((2, WIN), jnp.int32),
            pltpu.VMEM((2, REM), jnp.int32),
            pltpu.VMEM((NUP,), jnp.float32),
            pltpu.VMEM((NUP,), jnp.float32),
            pltpu.SemaphoreType.DMA,
            pltpu.SemaphoreType.DMA,
        ],
        compiler_params=pltpu.CompilerParams(
            use_tc_tiling_on_sc=True,
            needs_layout_passes=False,
        ),
    )


_dot = functools.partial(
    lax.dot_general,
    dimension_numbers=(((1,), (0,)), ((), ())),
    preferred_element_type=jnp.float32,
    precision=lax.Precision.HIGHEST,
)


def _dense_body(pu_ref, pi_ref, xu_ref, xi_ref, weu_ref, beu_ref, wei_ref,
                bei_ref, w1_ref, b1_ref, w2_ref, b2_ref, src_ref, dst_ref,
                out_ref):
    inv_e = jnp.float32(1.0 / E)
    cu = pu_ref[pl.ds(0, NU)]
    ci = pi_ref[pl.ds(0, NI)]
    for w in range(1, NW):
        cu = cu + pu_ref[pl.ds(w * NUP, NU)]
        ci = ci + pi_ref[pl.ds(w * NUP, NI)]

    mean_user = jnp.sum(xu_ref[...] * cu[:, None], axis=0,
                        keepdims=True) * inv_e
    mean_item = jnp.sum(xi_ref[...] * ci[:, None], axis=0,
                        keepdims=True) * inv_e

    msg_i1 = _dot(mean_user, weu_ref[...]) + beu_ref[...]
    msg_u1 = _dot(mean_item, wei_ref[...]) + bei_ref[...]

    p_u = lax.slice(ci, (0,), (1,)).reshape(1, 1) * inv_e
    p_i = lax.slice(cu, (0,), (1,)).reshape(1, 1) * inv_e
    scale = p_u * p_i
    u_vec = scale * jnp.maximum(msg_u1, 0.0)
    i_vec = scale * jnp.maximum(msg_i1, 0.0)

    z = jnp.zeros((1, H), jnp.float32)
    combos = jnp.concatenate(
        [
            jnp.concatenate([z, z], axis=1),
            jnp.concatenate([z, i_vec], axis=1),
            jnp.concatenate([u_vec, z], axis=1),
            jnp.concatenate([u_vec, i_vec], axis=1),
        ],
        axis=0,
    )
    hid = jnp.maximum(_dot(combos, w1_ref[...]) + b1_ref[...], 0.0)
    vals = jax.nn.sigmoid(_dot(hid, w2_ref[...]) + b2_ref[...])

    v00 = lax.slice(vals, (0, 0), (1, 1))
    v01 = lax.slice(vals, (1, 0), (2, 1))
    v10 = lax.slice(vals, (2, 0), (3, 1))
    v11 = lax.slice(vals, (3, 0), (4, 1))

    s_mask = src_ref[...] == 0
    d_mask = dst_ref[...] == 0
    out_ref[...] = jnp.where(
        s_mask,
        jnp.where(d_mask, v11, v10),
        jnp.where(d_mask, v01, v00),
    )


_dense = pl.pallas_call(
    _dense_body,
    out_shape=jax.ShapeDtypeStruct((EQ_ROWS, 128), jnp.float32),
)


def kernel(x_user, x_item, edge_index_u2i, edge_index_i2u, edge_label_index,
           W_emb_user, b_emb_user, W_emb_item, b_emb_item, W1, b1, W2, b2):
    part_u, part_i = _hist()(edge_index_u2i.astype(jnp.int32),
                             edge_index_i2u.astype(jnp.int32))

    eli = edge_label_index.astype(jnp.int32)
    pad = jnp.ones((2, EQ_PAD - EQ), jnp.int32)
    eli_p = jnp.concatenate([eli, pad], axis=1).reshape(2, EQ_ROWS, 128)

    out2d = _dense(
        part_u, part_i,
        x_user, x_item,
        W_emb_user, b_emb_user.reshape(1, H),
        W_emb_item, b_emb_item.reshape(1, H),
        W1, b1.reshape(1, H),
        W2, b2.reshape(1, 1),
        eli_p[0], eli_p[1],
    )
    return out2d.reshape(EQ_PAD)[:EQ]


# speedup vs baseline: 1.9869x; 1.0053x over previous
import functools

import jax
import jax.numpy as jnp
from jax import lax
from jax.experimental import pallas as pl
from jax.experimental.pallas import tpu as pltpu
from jax.experimental.pallas import tpu_sc as plsc

NU = 10000
NI = 10000
E = 320000
EQ = 100000
D = 128
H = 64

NC = 2
NS = 16
NW = NC * NS
L = 16
CHUNK = E // NW
NUP = 10240

EQ_ROWS = 782
EQ_PAD = EQ_ROWS * 128


WIN = 9984
REM = E - NW * WIN


def _hist_body(eu_hbm, ei_hbm, out_u, out_i, idx_u, idx_i, rem_v, cu_v, ci_v,
               sem_u, sem_i):
    wid = lax.axis_index("c") * NS + lax.axis_index("s")
    zeros16 = jnp.zeros((L,), jnp.float32)
    ones16 = jnp.ones((L,), jnp.float32)

    base = wid * WIN
    cp_u = pltpu.async_copy(eu_hbm.at[:, pl.ds(base, WIN)], idx_u, sem_u)
    cp_i = pltpu.async_copy(ei_hbm.at[:, pl.ds(base, WIN)], idx_i, sem_i)

    def zero_body(i, carry):
        cu_v[pl.ds(i * L, L)] = zeros16
        ci_v[pl.ds(i * L, L)] = zeros16
        return carry

    lax.fori_loop(0, NUP // L, zero_body, 0, unroll=8)

    cp_u.wait()

    @plsc.parallel_loop(0, WIN // L, unroll=8)
    def _(i):
        iv = idx_u[0, pl.ds(i * L, L)]
        plsc.addupdate_scatter(cu_v, [iv], ones16)

    cp_i.wait()

    @plsc.parallel_loop(0, WIN // L, unroll=8)
    def _(i):
        iv = idx_i[0, pl.ds(i * L, L)]
        plsc.addupdate_scatter(ci_v, [iv], ones16)

    @pl.when(wid == 0)
    def _():
        pltpu.sync_copy(eu_hbm.at[:, pl.ds(NW * WIN, REM)], rem_v)

        def add_ru(i, carry):
            iv = rem_v[0, pl.ds(i * L, L)]
            plsc.addupdate_scatter(cu_v, [iv], ones16)
            return carry

        lax.fori_loop(0, REM // L, add_ru, 0, unroll=4)

        pltpu.sync_copy(ei_hbm.at[:, pl.ds(NW * WIN, REM)], rem_v)

        def add_ri(i, carry):
            iv = rem_v[0, pl.ds(i * L, L)]
            plsc.addupdate_scatter(ci_v, [iv], ones16)
            return carry

        lax.fori_loop(0, REM // L, add_ri, 0, unroll=4)

    pltpu.sync_copy(cu_v, out_u.at[pl.ds(wid * NUP, NUP)])
    pltpu.sync_copy(ci_v, out_i.at[pl.ds(wid * NUP, NUP)])


@functools.cache
def _hist():
    return pl.kernel(
        _hist_body,
        mesh=plsc.VectorSubcoreMesh(core_axis_name="c", subcore_axis_name="s"),
        out_type=[
            jax.ShapeDtypeStruct((NW * NUP,), jnp.float32),
            jax.ShapeDtypeStruct((NW * NUP,), jnp.float32),
        ],
        scratch_types=[
            pltpu.VMEM((2, WIN), jnp.int32),
            pltpu.VMEM((2, WIN), jnp.int32),
            pltpu.VMEM((2, REM), jnp.int32),
            pltpu.VMEM((NUP,), jnp.float32),
            pltpu.VMEM((NUP,), jnp.float32),
            pltpu.SemaphoreType.DMA,
            pltpu.SemaphoreType.DMA,
        ],
        compiler_params=pltpu.CompilerParams(
            use_tc_tiling_on_sc=True,
            needs_layout_passes=False,
        ),
    )


_dot = functools.partial(
    lax.dot_general,
    dimension_numbers=(((1,), (0,)), ((), ())),
    preferred_element_type=jnp.float32,
    precision=lax.Precision.HIGHEST,
)


def _dense_body(pu_ref, pi_ref, xu_ref, xi_ref, weu_ref, beu_ref, wei_ref,
                bei_ref, w1_ref, b1_ref, w2_ref, b2_ref, src_ref, dst_ref,
                out_ref):
    inv_e = jnp.float32(1.0 / E)
    cu = pu_ref[pl.ds(0, NU)]
    ci = pi_ref[pl.ds(0, NI)]
    for w in range(1, NW):
        cu = cu + pu_ref[pl.ds(w * NUP, NU)]
        ci = ci + pi_ref[pl.ds(w * NUP, NI)]

    mean_user = jnp.sum(xu_ref[...] * cu[:, None], axis=0,
                        keepdims=True) * inv_e
    mean_item = jnp.sum(xi_ref[...] * ci[:, None], axis=0,
                        keepdims=True) * inv_e

    msg_i1 = _dot(mean_user, weu_ref[...]) + beu_ref[...]
    msg_u1 = _dot(mean_item, wei_ref[...]) + bei_ref[...]

    p_u = lax.slice(ci, (0,), (1,)).reshape(1, 1) * inv_e
    p_i = lax.slice(cu, (0,), (1,)).reshape(1, 1) * inv_e
    scale = p_u * p_i
    u_vec = scale * jnp.maximum(msg_u1, 0.0)
    i_vec = scale * jnp.maximum(msg_i1, 0.0)

    z = jnp.zeros((1, H), jnp.float32)
    combos = jnp.concatenate(
        [
            jnp.concatenate([z, z], axis=1),
            jnp.concatenate([z, i_vec], axis=1),
            jnp.concatenate([u_vec, z], axis=1),
            jnp.concatenate([u_vec, i_vec], axis=1),
        ],
        axis=0,
    )
    hid = jnp.maximum(_dot(combos, w1_ref[...]) + b1_ref[...], 0.0)
    vals = jax.nn.sigmoid(_dot(hid, w2_ref[...]) + b2_ref[...])

    v00 = lax.slice(vals, (0, 0), (1, 1))
    v01 = lax.slice(vals, (1, 0), (2, 1))
    v10 = lax.slice(vals, (2, 0), (3, 1))
    v11 = lax.slice(vals, (3, 0), (4, 1))

    s_mask = src_ref[...] == 0
    d_mask = dst_ref[...] == 0
    out_ref[...] = jnp.where(
        s_mask,
        jnp.where(d_mask, v11, v10),
        jnp.where(d_mask, v01, v00),
    )


_dense = pl.pallas_call(
    _dense_body,
    out_shape=jax.ShapeDtypeStruct((EQ_ROWS, 128), jnp.float32),
)


def kernel(x_user, x_item, edge_index_u2i, edge_index_i2u, edge_label_index,
           W_emb_user, b_emb_user, W_emb_item, b_emb_item, W1, b1, W2, b2):
    part_u, part_i = _hist()(edge_index_u2i.astype(jnp.int32),
                             edge_index_i2u.astype(jnp.int32))

    eli = edge_label_index.astype(jnp.int32)
    pad = jnp.ones((2, EQ_PAD - EQ), jnp.int32)
    eli_p = jnp.concatenate([eli, pad], axis=1).reshape(2, EQ_ROWS, 128)

    out2d = _dense(
        part_u, part_i,
        x_user, x_item,
        W_emb_user, b_emb_user.reshape(1, H),
        W_emb_item, b_emb_item.reshape(1, H),
        W1, b1.reshape(1, H),
        W2, b2.reshape(1, 1),
        eli_p[0], eli_p[1],
    )
    return out2d.reshape(EQ_PAD)[:EQ]
